# Initial kernel scaffold; baseline (speedup 1.0000x reference)
import jax
import jax.numpy as jnp
from jax.experimental import pallas as pl


def kernel(x, mask):
    raise NotImplementedError("write your pallas kernel here")


# speedup vs baseline: 1.0392x; 1.0392x over previous
import functools

import jax
import jax.numpy as jnp
from jax import lax
from jax.experimental import pallas as pl
from jax.experimental.pallas import tpu as pltpu
from jax.experimental.pallas import tpu_sc as plsc

B = 4
S = 8192
S1 = S + 1
D = 1024
K = int(S * (1 - 0.1))
P = K + 1
C = 128
NCH = S // C
GSEG = 8200
GTOT = B * GSEG + 8
CH = 16
NCHUNK = P // CH + 1
WPB = 8
CPW = -(-NCHUNK // WPB)
GWIN = CPW * CH

_sc_mesh = plsc.VectorSubcoreMesh(core_axis_name="c", subcore_axis_name="s")


def _rank_body(mt_ref, m4_ref, out_ref):
    b = pl.program_id(0)
    ic = pl.program_id(1)
    vi = m4_ref[...].reshape(1, C)
    mt = mt_ref[...]
    bsel = lax.broadcasted_iota(jnp.int32, (1, B), 1) == b
    col = jnp.sum(jnp.where(bsel, mt, 0.0), axis=1, keepdims=True)
    gt = col > vi
    eq = col == vi
    jio = lax.broadcasted_iota(jnp.int32, (S, C), 0)
    iio = lax.broadcasted_iota(jnp.int32, (1, C), 1) + ic * C
    one, zero = jnp.int32(1), jnp.int32(0)
    ones = jnp.where(eq, jnp.where(jio < iio, one, zero),
                     jnp.where(gt, one, zero))
    cnt = jnp.sum(ones, axis=0, keepdims=True)
    out_ref[...] = cnt.reshape(1, 1, 1, C)


_rank = pl.pallas_call(
    _rank_body,
    grid=(B, NCH),
    in_specs=[
        pl.BlockSpec((S, B), lambda b, ic: (0, 0)),
        pl.BlockSpec((1, 1, 1, C), lambda b, ic: (b, ic, 0, 0)),
    ],
    out_specs=pl.BlockSpec((1, 1, 1, C), lambda b, ic: (b, ic, 0, 0)),
    out_shape=jax.ShapeDtypeStruct((B, NCH, 1, C), jnp.int32),
)


@functools.partial(
    pl.kernel,
    out_type=jax.ShapeDtypeStruct((GTOT,), jnp.int32),
    mesh=_sc_mesh,
    scratch_types=[
        pltpu.VMEM((1024,), jnp.int32),
        pltpu.VMEM((8, 128), jnp.int32),
        pltpu.VMEM((8, 128), jnp.int32),
        pltpu.VMEM((2, 16), jnp.int32),
        pltpu.SemaphoreType.DMA,
    ],
)
def _invert(rk_hbm, g_hbm, rk_v, pos_v, val_v, cls_v, sem):
    wid = lax.axis_index("s") * 2 + lax.axis_index("c")
    b = wid // WPB
    wi = lax.rem(wid, WPB)
    pltpu.sync_copy(rk_hbm.at[pl.ds(b * S + wi * 1024, 1024)], rk_v)
    lane = lax.broadcasted_iota(jnp.int32, (16,), 0)
    for j in range(64):
        r = rk_v[pl.ds(j * 16, 16)]
        pos_v[j // 8, pl.ds((j % 8) * 16, 16)] = r + (b * GSEG + 1)
        val_v[j // 8, pl.ds((j % 8) * 16, 16)] = (
            wi * 1024 + j * 16 + 1) + lane
    cps = [pltpu.async_copy(val_v.at[t], g_hbm.at[pos_v.at[t]], sem)
           for t in range(8)]
    for cp in cps:
        cp.wait()

    @pl.when(wid == 0)
    def _():
        cls_v[0, :] = jnp.where(lane < B, lane * GSEG, GTOT - 8)
        cls_v[1, :] = lane * 0
        pltpu.async_copy(cls_v.at[1], g_hbm.at[cls_v.at[0]], sem).wait()


@functools.partial(
    pl.kernel,
    out_type=jax.ShapeDtypeStruct((B, P, D), jnp.float32),
    mesh=_sc_mesh,
    scratch_types=[
        pltpu.VMEM((GWIN,), jnp.int32),
        pltpu.VMEM((CH, D), jnp.float32),
        pltpu.SemaphoreType.DMA,
        pltpu.SemaphoreType.DMA,
    ],
)
def _gather(x_hbm, g_hbm, out_hbm, gwin_v, rows_v, isem, osem):
    wid = lax.axis_index("s") * 2 + lax.axis_index("c")
    b = wid // WPB
    wi = lax.rem(wid, WPB)
    pltpu.sync_copy(g_hbm.at[pl.ds(b * GSEG + wi * GWIN, GWIN)], gwin_v)
    nch = jnp.where(wi == WPB - 1, NCHUNK - (WPB - 1) * CPW, CPW)
    lane = lax.broadcasted_iota(jnp.int32, (16,), 0)

    def body(lc, carry):
        c = wi * CPW + lc
        base = jnp.minimum(c * CH, P - CH)
        idx = gwin_v[pl.ds(base - wi * GWIN, CH)]
        pltpu.async_copy(x_hbm.at[b].at[idx], rows_v, isem).wait()
        pltpu.async_copy(rows_v, out_hbm.at[b].at[base + lane], osem).wait()
        return carry

    lax.fori_loop(0, nch, body, 0)


def kernel(x, mask):
    ranks4 = _rank(mask.T, mask.reshape(B, NCH, 1, C))
    g = _invert(ranks4.reshape(B * S))
    return _gather(x, g)


# speedup vs baseline: 1.1580x; 1.1143x over previous
import functools

import jax
import jax.numpy as jnp
from jax import lax
from jax.experimental import pallas as pl
from jax.experimental.pallas import tpu as pltpu
from jax.experimental.pallas import tpu_sc as plsc

B = 4
S = 8192
S1 = S + 1
D = 1024
K = int(S * (1 - 0.1))
P = K + 1
C = 128
NCH = S // C
GSEG = 8200
GTOT = B * GSEG + 8
CH = 16
NCHUNK = P // CH + 1
WPB = 8
CPW = -(-NCHUNK // WPB)
GWIN = CPW * CH + CH

_sc_mesh = plsc.VectorSubcoreMesh(core_axis_name="c", subcore_axis_name="s")


def _rank_body(mt_ref, m4_ref, out_ref):
    b = pl.program_id(0)
    ic = pl.program_id(1)
    vi = m4_ref[...].reshape(1, C)
    mt = mt_ref[...]
    bsel = lax.broadcasted_iota(jnp.int32, (1, B), 1) == b
    col = jnp.sum(jnp.where(bsel, mt, 0.0), axis=1, keepdims=True)
    gt = col > vi
    eq = col == vi
    jio = lax.broadcasted_iota(jnp.int32, (S, C), 0)
    iio = lax.broadcasted_iota(jnp.int32, (1, C), 1) + ic * C
    one, zero = jnp.int32(1), jnp.int32(0)
    ones = jnp.where(eq, jnp.where(jio < iio, one, zero),
                     jnp.where(gt, one, zero))
    cnt = jnp.sum(ones, axis=0, keepdims=True)
    out_ref[...] = cnt.reshape(1, 1, 1, C)


_rank = pl.pallas_call(
    _rank_body,
    grid=(B, NCH),
    in_specs=[
        pl.BlockSpec((S, B), lambda b, ic: (0, 0)),
        pl.BlockSpec((1, 1, 1, C), lambda b, ic: (b, ic, 0, 0)),
    ],
    out_specs=pl.BlockSpec((1, 1, 1, C), lambda b, ic: (b, ic, 0, 0)),
    out_shape=jax.ShapeDtypeStruct((B, NCH, 1, C), jnp.int32),
)


NB = 6
NCW = 60


@functools.partial(
    pl.kernel,
    out_type=jax.ShapeDtypeStruct((B, P, D), jnp.float32),
    mesh=_sc_mesh,
    compiler_params=pltpu.CompilerParams(needs_layout_passes=False),
    scratch_types=[
        pltpu.VMEM((S,), jnp.int32),
        pltpu.VMEM((GWIN,), jnp.int32),
        pltpu.VMEM((NB, CH, D), jnp.float32),
        pltpu.SemaphoreType.DMA((NB,)),
        pltpu.SemaphoreType.DMA((NB,)),
    ],
)
def _topk_gather(rk_hbm, x_hbm, out_hbm, rk_v, gwin_v, rows_v, isems, osems):
    wid = lax.axis_index("s") * 2 + lax.axis_index("c")
    b = wid // WPB
    wi = lax.rem(wid, WPB)
    r0 = (wi * P) // WPB
    r1 = ((wi + 1) * P) // WPB
    wlen = r1 - r0
    pltpu.sync_copy(rk_hbm.at[pl.ds(b * S, S)], rk_v)
    lane = lax.broadcasted_iota(jnp.int32, (16,), 0)

    def inv_body(j8, carry):
        for s8 in range(8):
            jj = j8 * 8 + s8
            r = rk_v[pl.ds(jj * 16, 16)]
            pos = r + (1 - r0)
            m = jnp.logical_and(pos >= 0, pos < wlen)
            posc = jnp.where(m, pos, GWIN - 16 + lane)
            plsc.store_scatter(gwin_v, [posc], jj * 16 + 1 + lane)
        return carry

    lax.fori_loop(0, 64, inv_body, 0)

    @pl.when(wi == 0)
    def _():
        v0 = gwin_v[pl.ds(0, 16)]
        gwin_v[pl.ds(0, 16)] = jnp.where(lane == 0, 0, v0)

    def start_g(c, k):
        base = jnp.minimum(r0 + c * CH, r1 - CH)
        idx = gwin_v[pl.ds(base - r0, CH)]
        cp = pltpu.async_copy(x_hbm.at[b].at[idx], rows_v.at[k], isems.at[k])
        return cp, base

    gd = [None] * NCW
    gb = [None] * NCW
    od = [None] * NCW
    for c in range(NB):
        gd[c], gb[c] = start_g(c, c)
    for c in range(NCW):
        gd[c].wait()
        od[c] = pltpu.async_copy(
            rows_v.at[c % NB], out_hbm.at[b].at[gb[c] + lane],
            osems.at[c % NB])
        if c + NB < NCW:
            od[c].wait()
            gd[c + NB], gb[c + NB] = start_g(c + NB, c % NB)
    for c in range(NCW - NB, NCW):
        od[c].wait()


def kernel(x, mask):
    ranks4 = _rank(mask.T, mask.reshape(B, NCH, 1, C))
    return _topk_gather(ranks4.reshape(B * S), x)


# speedup vs baseline: 2.2864x; 1.9745x over previous
import functools

import jax
import jax.numpy as jnp
from jax import lax
from jax.experimental import pallas as pl
from jax.experimental.pallas import tpu as pltpu
from jax.experimental.pallas import tpu_sc as plsc

B = 4
S = 8192
S1 = S + 1
D = 1024
K = int(S * (1 - 0.1))
P = K + 1
R = 64
L = 128
NBITS = 13

CH = 16
WPB = 8
NCW = 58
NB = 6
WLEN = 944

_sc_mesh = plsc.VectorSubcoreMesh(core_axis_name="c", subcore_axis_name="s")


def _sort_body(m_ref, out_ref):
    key = m_ref[0]
    rows = lax.broadcasted_iota(jnp.int32, (R, L), 0)
    lanes = lax.broadcasted_iota(jnp.int32, (R, L), 1)
    e = rows * L + lanes
    idx = e
    mcache = {}

    def bitmask(bit):
        if bit not in mcache:
            if bit >= S:
                mcache[bit] = None
            elif bit < L:
                mcache[bit] = (lanes & bit) == 0
            else:
                mcache[bit] = (rows & (bit >> 7)) == 0
        return mcache[bit]

    for kb in range(1, NBITS + 1):
        fwd = bitmask(1 << kb)
        for jb in range(kb - 1, -1, -1):
            d = 1 << jb
            lo = bitmask(d)
            ax, sh, n = (1, d, L) if d < L else (0, d >> 7, R)
            pk = jnp.where(lo, pltpu.roll(key, n - sh, ax),
                           pltpu.roll(key, sh, ax))
            pi = jnp.where(lo, pltpu.roll(idx, n - sh, ax),
                           pltpu.roll(idx, sh, ax))
            mb = (key > pk) | ((key == pk) & (idx < pi))
            x1 = jnp.logical_xor(mb, lo)
            keep = jnp.logical_not(x1) if fwd is None \
                else jnp.logical_xor(x1, fwd)
            key = jnp.where(keep, key, pk)
            idx = jnp.where(keep, idx, pi)

    rolled = pltpu.roll(idx, 1, 1)
    rowr = pltpu.roll(rolled, 1, 0)
    shifted = jnp.where(lanes == 0, rowr, rolled)
    out_ref[0] = jnp.where(e == 0, 0, shifted + 1)


_sort = pl.pallas_call(
    _sort_body,
    grid=(B,),
    in_specs=[pl.BlockSpec((1, R, L), lambda b: (b, 0, 0))],
    out_specs=pl.BlockSpec((1, R, L), lambda b: (b, 0, 0)),
    out_shape=jax.ShapeDtypeStruct((B, R, L), jnp.int32),
)


@functools.partial(
    pl.kernel,
    out_type=jax.ShapeDtypeStruct((B, P, D), jnp.float32),
    mesh=_sc_mesh,
    compiler_params=pltpu.CompilerParams(needs_layout_passes=False),
    scratch_types=[
        pltpu.VMEM((WLEN,), jnp.int32),
        pltpu.VMEM((NB, CH, D), jnp.float32),
        pltpu.SemaphoreType.DMA((NB,)),
        pltpu.SemaphoreType.DMA((NB,)),
    ],
)
def _topk_gather(g_hbm, x_hbm, out_hbm, gwin_v, rows_v, isems, osems):
    wid = lax.axis_index("s") * 2 + lax.axis_index("c")
    b = wid // WPB
    wi = lax.rem(wid, WPB)
    r0 = (wi * P) // WPB
    r1 = ((wi + 1) * P) // WPB
    g0 = b * S + r0
    al8 = g0 // 8
    off0 = g0 - al8 * 8
    pltpu.sync_copy(g_hbm.at[pl.ds(al8 * 8, WLEN)], gwin_v)
    lane = lax.broadcasted_iota(jnp.int32, (16,), 0)

    def start_g(c, k):
        base = jnp.minimum(r0 + c * CH, r1 - CH)
        idx = gwin_v[pl.ds(off0 + base - r0, CH)]
        cp = pltpu.async_copy(x_hbm.at[b].at[idx], rows_v.at[k], isems.at[k])
        return cp, base

    gd = [None] * NCW
    gb = [None] * NCW
    od = [None] * NCW
    for c in range(NB):
        gd[c], gb[c] = start_g(c, c)
    for c in range(NCW):
        gd[c].wait()
        od[c] = pltpu.async_copy(
            rows_v.at[c % NB], out_hbm.at[b].at[gb[c] + lane],
            osems.at[c % NB])
        if c + NB < NCW:
            od[c].wait()
            gd[c + NB], gb[c + NB] = start_g(c + NB, c % NB)
    for c in range(NCW - NB, NCW):
        od[c].wait()


def kernel(x, mask):
    g = _sort(mask.reshape(B, R, L))
    return _topk_gather(g.reshape(B * S), x)
